# Initial kernel scaffold; baseline (speedup 1.0000x reference)
import functools

import jax
import jax.numpy as jnp
from jax import lax
from jax.experimental import pallas as pl
from jax.experimental.pallas import tpu as pltpu
from jax.experimental.pallas import tpu_sc as plsc

_N = 10000
_E = 320000
_D_IN = 128
_D_HID = 128
_D_OUT = 64

_NC = 2
_NS = 16
_N_PAD = 10240
_ROWS_PER_TILE = _N_PAD // _NS
_E_CORE = _E // _NC
_E_TILE = _E_CORE // _NS
_CHUNK = 128
_NFULL = _E_TILE // _CHUNK
_TAIL = _E_TILE - _NFULL * _CHUNK

_mesh = plsc.VectorSubcoreMesh(core_axis_name="c", subcore_axis_name="s")


@functools.partial(
    pl.kernel,
    out_type=jax.ShapeDtypeStruct((_NC, _N), jnp.float32),
    mesh=_mesh,
    scratch_types=[
        pltpu.VMEM((_CHUNK,), jnp.int32),
        pltpu.VMEM((_TAIL,), jnp.int32),
        pltpu.VMEM((_CHUNK,), jnp.float32),
        pltpu.VMEM((_ROWS_PER_TILE,), jnp.float32),
        pltpu.VMEM_SHARED((_N_PAD,), jnp.float32),
        pltpu.SemaphoreType.DMA,
    ],
)
def _deg_kernel(dst_hbm, out_hbm, idx_v, idxt_v, ones_v, zrow_v, acc, sem):
    cid = lax.axis_index("c")
    sid = lax.axis_index("s")

    @pl.loop(0, _ROWS_PER_TILE // 16)
    def _(i):
        zrow_v[pl.ds(i * 16, 16)] = jnp.zeros((16,), jnp.float32)

    @pl.loop(0, _CHUNK // 16)
    def _(i):
        ones_v[pl.ds(i * 16, 16)] = jnp.full((16,), 1.0, jnp.float32)

    rbase = sid * _ROWS_PER_TILE
    pltpu.sync_copy(zrow_v, acc.at[pl.ds(rbase, _ROWS_PER_TILE)])
    plsc.subcore_barrier()

    ebase = cid * _E_CORE + sid * _E_TILE

    @pl.loop(0, _NFULL)
    def _(j):
        pltpu.sync_copy(dst_hbm.at[pl.ds(ebase + j * _CHUNK, _CHUNK)], idx_v)
        pltpu.sync_copy(ones_v, acc.at[idx_v], add=True)

    pltpu.sync_copy(dst_hbm.at[pl.ds(ebase + _NFULL * _CHUNK, _TAIL)], idxt_v)
    pltpu.sync_copy(ones_v.at[pl.ds(0, _TAIL)], acc.at[idxt_v], add=True)
    plsc.subcore_barrier()

    @pl.when(sid < _NS - 1)
    def _():
        pltpu.sync_copy(acc.at[pl.ds(rbase, _ROWS_PER_TILE)],
                        out_hbm.at[cid, pl.ds(rbase, _ROWS_PER_TILE)])

    @pl.when(sid == _NS - 1)
    def _():
        pltpu.sync_copy(acc.at[pl.ds((_NS - 1) * _ROWS_PER_TILE,
                                     _N - (_NS - 1) * _ROWS_PER_TILE)],
                        out_hbm.at[cid, pl.ds((_NS - 1) * _ROWS_PER_TILE,
                                              _N - (_NS - 1) * _ROWS_PER_TILE)])


def _make_agg(d):
    last = _N - (_NS - 1) * _ROWS_PER_TILE

    @functools.partial(
        pl.kernel,
        out_type=jax.ShapeDtypeStruct((_NC, _N, d), jnp.float32),
        mesh=_mesh,
        scratch_types=[
            pltpu.VMEM((_CHUNK,), jnp.int32),
            pltpu.VMEM((_CHUNK,), jnp.int32),
            pltpu.VMEM((_TAIL,), jnp.int32),
            pltpu.VMEM((_TAIL,), jnp.int32),
            pltpu.VMEM((_CHUNK, d), jnp.float32),
            pltpu.VMEM((_TAIL, d), jnp.float32),
            pltpu.VMEM_SHARED((_N_PAD, d), jnp.float32),
            pltpu.SemaphoreType.DMA,
        ],
    )
    def _agg(src_hbm, dst_hbm, hs_hbm, out_hbm,
             sidx, didx, sidxt, didxt, rows, rowst, acc, sem):
        cid = lax.axis_index("c")
        sid = lax.axis_index("s")

        @pl.loop(0, _CHUNK)
        def _(r):
            @pl.loop(0, d // 16)
            def _(q):
                rows[r, pl.ds(q * 16, 16)] = jnp.zeros((16,), jnp.float32)

        rbase = sid * _ROWS_PER_TILE

        @pl.loop(0, _ROWS_PER_TILE // _CHUNK)
        def _(k):
            pltpu.sync_copy(rows, acc.at[pl.ds(rbase + k * _CHUNK, _CHUNK)])

        plsc.subcore_barrier()

        ebase = cid * _E_CORE + sid * _E_TILE

        @pl.loop(0, _NFULL)
        def _(j):
            pltpu.sync_copy(src_hbm.at[pl.ds(ebase + j * _CHUNK, _CHUNK)], sidx)
            pltpu.sync_copy(dst_hbm.at[pl.ds(ebase + j * _CHUNK, _CHUNK)], didx)
            pltpu.async_copy(hs_hbm.at[sidx], rows, sem).wait()
            pltpu.sync_copy(rows, acc.at[didx], add=True)

        tbase = ebase + _NFULL * _CHUNK
        pltpu.sync_copy(src_hbm.at[pl.ds(tbase, _TAIL)], sidxt)
        pltpu.sync_copy(dst_hbm.at[pl.ds(tbase, _TAIL)], didxt)
        pltpu.async_copy(hs_hbm.at[sidxt], rowst, sem).wait()
        pltpu.sync_copy(rowst, acc.at[didxt], add=True)
        plsc.subcore_barrier()

        @pl.when(sid < _NS - 1)
        def _():
            pltpu.sync_copy(acc.at[pl.ds(rbase, _ROWS_PER_TILE)],
                            out_hbm.at[cid, pl.ds(rbase, _ROWS_PER_TILE)])

        @pl.when(sid == _NS - 1)
        def _():
            pltpu.sync_copy(acc.at[pl.ds((_NS - 1) * _ROWS_PER_TILE, last)],
                            out_hbm.at[cid, pl.ds((_NS - 1) * _ROWS_PER_TILE,
                                                  last)])

    return _agg


_agg_hid = _make_agg(_D_HID)
_agg_out = _make_agg(_D_OUT)

_BLK = 1000
_GRID = _N // _BLK


def _pre_body(deg_ref, x_ref, w1_ref, dis_ref, hs1_ref):
    deg = deg_ref[0, :] + deg_ref[1, :] + 1.0
    dis = lax.rsqrt(deg)[:, None]
    h = jnp.dot(x_ref[...], w1_ref[...], preferred_element_type=jnp.float32)
    dis_ref[...] = dis
    hs1_ref[...] = h * dis


def _pre_call(degp, x, w1):
    return pl.pallas_call(
        _pre_body,
        grid=(_GRID,),
        in_specs=[
            pl.BlockSpec((_NC, _BLK), lambda i: (0, i)),
            pl.BlockSpec((_BLK, _D_IN), lambda i: (i, 0)),
            pl.BlockSpec((_D_IN, _D_HID), lambda i: (0, 0)),
        ],
        out_specs=[
            pl.BlockSpec((_BLK, 1), lambda i: (i, 0)),
            pl.BlockSpec((_BLK, _D_HID), lambda i: (i, 0)),
        ],
        out_shape=[
            jax.ShapeDtypeStruct((_N, 1), jnp.float32),
            jax.ShapeDtypeStruct((_N, _D_HID), jnp.float32),
        ],
    )(degp, x, w1)


def _mid_body(p1_ref, hs1_ref, dis_ref, b1_ref, w2_ref, hs2_ref):
    dis = dis_ref[...]
    p1 = p1_ref[0] + p1_ref[1] + hs1_ref[...]
    h1 = jnp.maximum(dis * p1 + b1_ref[...], 0.0)
    h2 = jnp.dot(h1, w2_ref[...], preferred_element_type=jnp.float32)
    hs2_ref[...] = h2 * dis


def _mid_call(p1, hs1, dis, b1, w2):
    return pl.pallas_call(
        _mid_body,
        grid=(_GRID,),
        in_specs=[
            pl.BlockSpec((_NC, _BLK, _D_HID), lambda i: (0, i, 0)),
            pl.BlockSpec((_BLK, _D_HID), lambda i: (i, 0)),
            pl.BlockSpec((_BLK, 1), lambda i: (i, 0)),
            pl.BlockSpec((1, _D_HID), lambda i: (0, 0)),
            pl.BlockSpec((_D_HID, _D_OUT), lambda i: (0, 0)),
        ],
        out_specs=pl.BlockSpec((_BLK, _D_OUT), lambda i: (i, 0)),
        out_shape=jax.ShapeDtypeStruct((_N, _D_OUT), jnp.float32),
    )(p1, hs1, dis, b1, w2)


def _post_body(p2_ref, hs2_ref, dis_ref, b2_ref, out_ref):
    o = dis_ref[...] * (p2_ref[0] + p2_ref[1] + hs2_ref[...]) + b2_ref[...]
    m = jnp.max(o, axis=1, keepdims=True)
    lse = m + jnp.log(jnp.sum(jnp.exp(o - m), axis=1, keepdims=True))
    out_ref[...] = o - lse


def _post_call(p2, hs2, dis, b2):
    return pl.pallas_call(
        _post_body,
        grid=(_GRID,),
        in_specs=[
            pl.BlockSpec((_NC, _BLK, _D_OUT), lambda i: (0, i, 0)),
            pl.BlockSpec((_BLK, _D_OUT), lambda i: (i, 0)),
            pl.BlockSpec((_BLK, 1), lambda i: (i, 0)),
            pl.BlockSpec((1, _D_OUT), lambda i: (0, 0)),
        ],
        out_specs=pl.BlockSpec((_BLK, _D_OUT), lambda i: (i, 0)),
        out_shape=jax.ShapeDtypeStruct((_N, _D_OUT), jnp.float32),
    )(p2, hs2, dis, b2)


def kernel(x, edge_index, W1, b1, W2, b2):
    src = edge_index[0]
    dst = edge_index[1]
    degp = _deg_kernel(dst)
    dis, hs1 = _pre_call(degp, x, W1)
    p1 = _agg_hid(src, dst, hs1)
    hs2 = _mid_call(p1, hs1, dis, b1[None, :], W2)
    p2 = _agg_out(src, dst, hs2)
    return _post_call(p2, hs2, dis, b2[None, :])


# speedup vs baseline: 16.0026x; 16.0026x over previous
import functools

import jax
import jax.numpy as jnp
from jax import lax
from jax.experimental import pallas as pl
from jax.experimental.pallas import tpu as pltpu
from jax.experimental.pallas import tpu_sc as plsc

_N = 10000
_E = 320000
_D_IN = 128
_D_HID = 128
_D_OUT = 64

_NC = 2
_NS = 16
_N_PAD = 10240
_ROWS_PER_TILE = _N_PAD // _NS
_E_CORE = _E // _NC
_E_TILE = _E_CORE // _NS
_CHUNK = 128
_NFULL = _E_TILE // _CHUNK
_TAIL = _E_TILE - _NFULL * _CHUNK

_mesh = plsc.VectorSubcoreMesh(core_axis_name="c", subcore_axis_name="s")


@functools.partial(
    pl.kernel,
    out_type=jax.ShapeDtypeStruct((_NC, _N_PAD), jnp.float32),
    mesh=_mesh,
    scratch_types=[
        pltpu.VMEM((_CHUNK,), jnp.int32),
        pltpu.VMEM((_TAIL,), jnp.int32),
        pltpu.VMEM((_CHUNK,), jnp.float32),
        pltpu.VMEM((_ROWS_PER_TILE,), jnp.float32),
        pltpu.VMEM_SHARED((_N_PAD,), jnp.float32),
        pltpu.SemaphoreType.DMA,
    ],
)
def _deg_kernel(dst_hbm, out_hbm, idx_v, idxt_v, ones_v, zrow_v, acc, sem):
    cid = lax.axis_index("c")
    sid = lax.axis_index("s")

    @pl.loop(0, _ROWS_PER_TILE // 16)
    def _(i):
        zrow_v[pl.ds(i * 16, 16)] = jnp.zeros((16,), jnp.float32)

    @pl.loop(0, _CHUNK // 16)
    def _(i):
        ones_v[pl.ds(i * 16, 16)] = jnp.full((16,), 1.0, jnp.float32)

    rbase = sid * _ROWS_PER_TILE
    pltpu.sync_copy(zrow_v, acc.at[pl.ds(rbase, _ROWS_PER_TILE)])
    plsc.subcore_barrier()

    ebase = cid * _E_CORE + sid * _E_TILE

    @pl.loop(0, _NFULL)
    def _(j):
        pltpu.sync_copy(dst_hbm.at[pl.ds(ebase + j * _CHUNK, _CHUNK)], idx_v)
        pltpu.sync_copy(ones_v, acc.at[idx_v], add=True)

    pltpu.sync_copy(dst_hbm.at[pl.ds(ebase + _NFULL * _CHUNK, _TAIL)], idxt_v)
    pltpu.sync_copy(ones_v.at[pl.ds(0, _TAIL)], acc.at[idxt_v], add=True)
    plsc.subcore_barrier()

    pltpu.sync_copy(acc.at[pl.ds(rbase, _ROWS_PER_TILE)],
                    out_hbm.at[cid, pl.ds(rbase, _ROWS_PER_TILE)])


def _make_agg(d):
    @functools.partial(
        pl.kernel,
        out_type=jax.ShapeDtypeStruct((_NC, _N_PAD, d), jnp.float32),
        mesh=_mesh,
        scratch_types=[
            pltpu.VMEM((_CHUNK,), jnp.int32),
            pltpu.VMEM((_CHUNK,), jnp.int32),
            pltpu.VMEM((_TAIL,), jnp.int32),
            pltpu.VMEM((_TAIL,), jnp.int32),
            pltpu.VMEM((_CHUNK, d), jnp.float32),
            pltpu.VMEM((_TAIL, d), jnp.float32),
            pltpu.VMEM_SHARED((_N_PAD, d), jnp.float32),
            pltpu.SemaphoreType.DMA,
        ],
    )
    def _agg(src_hbm, dst_hbm, hs_hbm, out_hbm,
             sidx, didx, sidxt, didxt, rows, rowst, acc, sem):
        cid = lax.axis_index("c")
        sid = lax.axis_index("s")

        @pl.loop(0, _CHUNK)
        def _(r):
            @pl.loop(0, d // 16)
            def _(q):
                rows[r, pl.ds(q * 16, 16)] = jnp.zeros((16,), jnp.float32)

        rbase = sid * _ROWS_PER_TILE

        @pl.loop(0, _ROWS_PER_TILE // _CHUNK)
        def _(k):
            pltpu.sync_copy(rows, acc.at[pl.ds(rbase + k * _CHUNK, _CHUNK)])

        plsc.subcore_barrier()

        ebase = cid * _E_CORE + sid * _E_TILE

        @pl.loop(0, _NFULL)
        def _(j):
            pltpu.sync_copy(src_hbm.at[pl.ds(ebase + j * _CHUNK, _CHUNK)], sidx)
            pltpu.sync_copy(dst_hbm.at[pl.ds(ebase + j * _CHUNK, _CHUNK)], didx)
            pltpu.async_copy(hs_hbm.at[sidx], rows, sem).wait()
            pltpu.sync_copy(rows, acc.at[didx], add=True)

        tbase = ebase + _NFULL * _CHUNK
        pltpu.sync_copy(src_hbm.at[pl.ds(tbase, _TAIL)], sidxt)
        pltpu.sync_copy(dst_hbm.at[pl.ds(tbase, _TAIL)], didxt)
        pltpu.async_copy(hs_hbm.at[sidxt], rowst, sem).wait()
        pltpu.sync_copy(rowst, acc.at[didxt], add=True)
        plsc.subcore_barrier()

        pltpu.sync_copy(acc.at[pl.ds(rbase, _ROWS_PER_TILE)],
                        out_hbm.at[cid, pl.ds(rbase, _ROWS_PER_TILE)])

    return _agg


_agg_hid = _make_agg(_D_HID)

_BLK = 1000
_GRID = _N // _BLK


def _pre_body(deg_ref, x_ref, w1_ref, dis_ref, hs1_ref):
    deg = deg_ref[0] + deg_ref[1] + 1.0
    dis = lax.rsqrt(deg)
    h = jnp.dot(x_ref[...], w1_ref[...], preferred_element_type=jnp.float32)
    dis_ref[...] = dis
    hs1_ref[...] = h * dis


def _pre_call(degp, x, w1):
    return pl.pallas_call(
        _pre_body,
        grid=(_GRID,),
        in_specs=[
            pl.BlockSpec((_NC, _BLK, 1), lambda i: (0, i, 0)),
            pl.BlockSpec((_BLK, _D_IN), lambda i: (i, 0)),
            pl.BlockSpec((_D_IN, _D_HID), lambda i: (0, 0)),
        ],
        out_specs=[
            pl.BlockSpec((_BLK, 1), lambda i: (i, 0)),
            pl.BlockSpec((_BLK, _D_HID), lambda i: (i, 0)),
        ],
        out_shape=[
            jax.ShapeDtypeStruct((_N, 1), jnp.float32),
            jax.ShapeDtypeStruct((_N, _D_HID), jnp.float32),
        ],
    )(degp, x, w1)


def _mid_body(p1_ref, hs1_ref, dis_ref, b1_ref, hsm_ref):
    dis = dis_ref[...]
    p1 = p1_ref[0] + p1_ref[1] + hs1_ref[...]
    h1 = jnp.maximum(dis * p1 + b1_ref[...], 0.0)
    hsm_ref[...] = h1 * dis


def _mid_call(p1, hs1, dis, b1):
    return pl.pallas_call(
        _mid_body,
        grid=(_GRID,),
        in_specs=[
            pl.BlockSpec((_NC, _BLK, _D_HID), lambda i: (0, i, 0)),
            pl.BlockSpec((_BLK, _D_HID), lambda i: (i, 0)),
            pl.BlockSpec((_BLK, 1), lambda i: (i, 0)),
            pl.BlockSpec((1, _D_HID), lambda i: (0, 0)),
        ],
        out_specs=pl.BlockSpec((_BLK, _D_HID), lambda i: (i, 0)),
        out_shape=jax.ShapeDtypeStruct((_N, _D_HID), jnp.float32),
    )(p1, hs1, dis, b1)


def _post_body(p2_ref, hsm_ref, dis_ref, w2_ref, b2_ref, out_ref):
    a = dis_ref[...] * (p2_ref[0] + p2_ref[1] + hsm_ref[...])
    o = jnp.dot(a, w2_ref[...], preferred_element_type=jnp.float32) \
        + b2_ref[...]
    m = jnp.max(o, axis=1, keepdims=True)
    lse = m + jnp.log(jnp.sum(jnp.exp(o - m), axis=1, keepdims=True))
    out_ref[...] = o - lse


def _post_call(p2, hsm, dis, w2, b2):
    return pl.pallas_call(
        _post_body,
        grid=(_GRID,),
        in_specs=[
            pl.BlockSpec((_NC, _BLK, _D_HID), lambda i: (0, i, 0)),
            pl.BlockSpec((_BLK, _D_HID), lambda i: (i, 0)),
            pl.BlockSpec((_BLK, 1), lambda i: (i, 0)),
            pl.BlockSpec((_D_HID, _D_OUT), lambda i: (0, 0)),
            pl.BlockSpec((1, _D_OUT), lambda i: (0, 0)),
        ],
        out_specs=pl.BlockSpec((_BLK, _D_OUT), lambda i: (i, 0)),
        out_shape=jax.ShapeDtypeStruct((_N, _D_OUT), jnp.float32),
    )(p2, hsm, dis, w2, b2)


def kernel(x, edge_index, W1, b1, W2, b2):
    src = edge_index[0]
    dst = edge_index[1]
    degp = _deg_kernel(dst)[:, :_N, None]
    dis, hs1 = _pre_call(degp, x, W1)
    p1 = _agg_hid(src, dst, hs1)[:, :_N]
    hsm = _mid_call(p1, hs1, dis, b1[None, :])
    p2 = _agg_hid(src, dst, hsm)[:, :_N]
    return _post_call(p2, hsm, dis, W2, b2[None, :])
